# baseline (speedup 1.0000x reference)
import functools

import numpy as np
import jax
import jax.numpy as jnp
from jax import lax
from jax.experimental import pallas as pl
from jax.experimental.pallas import tpu as pltpu
from jax.experimental.pallas import tpu_sc as plsc

L = 16
K = 256
N = 4096
R = 1024
NC = 2
NS = 16
NW = NC * NS
ROWS_PER_W = R // NW

_TOP = np.uint32(0x80000000)


def _to_key(f):
    b = plsc.bitcast(f, jnp.uint32)
    return jnp.where((b >> 31) == 1, ~b, b | _TOP)


def _from_key(k):
    b = jnp.where((k >> 31) == 1, k ^ _TOP, ~k)
    return plsc.bitcast(b, jnp.float32)


def _merge2_desc(v, w):
    rw = lax.rev(w, (0,))
    hi = jnp.maximum(v, rw)
    lo = jnp.minimum(v, rw)
    hi, _ = plsc.sort_key_val(hi, hi, descending=True)
    lo, _ = plsc.sort_key_val(lo, lo, descending=True)
    return hi, lo


def _popcount(m):
    return plsc.all_reduce_population_count(m)[0]


def _body(x_hbm, out_hbm, row_v0, row_v1, keys_v, bufA, bufB, bufC,
          out_stage, hist, hist12, sem):
    wid = lax.axis_index("c") * NS + lax.axis_index("s")
    iota = lax.iota(jnp.int32, L)
    zeros_i = jnp.zeros((L,), jnp.int32)
    ones_i = jnp.ones((L,), jnp.int32)

    def zero_hist(i, _):
        hist[pl.ds(i * L, L)] = zeros_i
        hist12[pl.ds((i % 256) * L, L)] = zeros_i
        return 0

    lax.fori_loop(0, 256, zero_hist, 0)

    def hist_pass(src, cnt, shift):
        @plsc.parallel_loop(0, (cnt + L - 1) // L, unroll=2)
        def _hp(i):
            k = src[pl.ds(i * L, L)]
            valid = iota < cnt - i * L
            d = ((k >> shift) & 0xFF).astype(jnp.int32)
            plsc.addupdate_scatter(hist, [d * L + iota], ones_i, mask=valid)

    def scan_hist(h, need):
        def gbody(jj, c):
            cum, found, gstar, cumg = c
            g = 15 - jj
            acc = h[pl.ds(g * 256, L)]
            for l in range(1, L):
                acc = acc + h[pl.ds(g * 256 + l * L, L)]
            gc = jnp.sum(acc)
            hit = jnp.logical_and(jnp.logical_not(found), cum + gc >= need)
            gstar = jnp.where(hit, g, gstar)
            cumg = jnp.where(hit, cum, cumg)
            found = jnp.logical_or(found, hit)
            cum = jnp.where(found, cum, cum + gc)
            return cum, found, gstar, cumg

        _, _, gstar, cumg = plsc.parallel_loop(
            0, 16, carry=(jnp.int32(0), jnp.bool_(False), jnp.int32(0),
                          jnp.int32(0)))(gbody)

        def ibody(l, c):
            cum, found, b, cnt = c
            d = gstar * 16 + 15 - l
            dc = jnp.sum(h[pl.ds(d * L, L)])
            hit = jnp.logical_and(jnp.logical_not(found), cum + dc >= need)
            b = jnp.where(hit, d, b)
            cnt = jnp.where(hit, dc, cnt)
            found = jnp.logical_or(found, hit)
            cum = jnp.where(found, cum, cum + dc)
            return cum, found, b, cnt

        cum_gt, _, b, cnt_eq = plsc.parallel_loop(
            0, 16, carry=(cumg, jnp.bool_(False), jnp.int32(0),
                          jnp.int32(0)))(ibody)
        return b, cum_gt, cnt_eq

    def compact_pass(src, dst, cnt, shift, bstar, offA):
        @plsc.parallel_loop(0, (cnt + L - 1) // L, unroll=2,
                            carry=(offA, jnp.int32(0)))
        def _cp(i, c):
            oA, oD = c
            k = src[pl.ds(i * L, L)]
            valid = iota < cnt - i * L
            d = ((k >> shift) & 0xFF).astype(jnp.int32)
            m_hi = jnp.logical_and(d > bstar, valid)
            m_eq = jnp.logical_and(d == bstar, valid)
            plsc.store_compressed(bufA.at[pl.ds(oA, L)], k, mask=m_hi)
            plsc.store_compressed(dst.at[pl.ds(oD, L)], k, mask=m_eq)
            plsc.store_scatter(hist, [d * L + iota], zeros_i, mask=valid)
            return oA + _popcount(m_hi), oD + _popcount(m_eq)

        return _cp

    def select_level(src, dst, shift, state):
        def skip(args):
            return args

        def run(args):
            def small(args2):
                offA, need, cnt = args2
                vs = []
                for j in range(3):
                    k = src[pl.ds(j * L, L)]
                    kv = jnp.where(iota + j * L < cnt, k, jnp.uint32(0))
                    ks, _ = plsc.sort_key_val(kv, kv, descending=True)
                    vs.append(ks)
                a0, a1 = _merge2_desc(vs[0], vs[1])
                zpad = jnp.zeros((L,), jnp.uint32)
                b0, b1r = lax.rev(zpad, (0,)), lax.rev(vs[2], (0,))
                t0 = jnp.maximum(a0, b0)
                t2 = jnp.minimum(a0, b0)
                t1 = jnp.maximum(a1, b1r)
                t3 = jnp.minimum(a1, b1r)
                outs = []
                for w in (jnp.maximum(t0, t1), jnp.minimum(t0, t1),
                          jnp.maximum(t2, t3)):
                    ws, _ = plsc.sort_key_val(w, w, descending=True)
                    outs.append(ws)
                for j in range(3):
                    plsc.store_compressed(bufA.at[pl.ds(offA + j * L, L)],
                                          outs[j],
                                          mask=iota < need - j * L)
                return offA + need, jnp.int32(0), jnp.int32(0)

            def big(args2):
                offA, need, cnt = args2
                hist_pass(src, cnt, shift)
                b, cum_gt, cnt_eq = scan_hist(hist, need)
                offA, _ = compact_pass(src, dst, cnt, shift, b, offA)
                return offA, need - cum_gt, cnt_eq

            return lax.cond(args[2] <= 3 * L, small, big, args)

        return lax.cond(state[2] > 0, run, skip, state)

    def process_row(buf, j):
        @plsc.parallel_loop(0, N // L, unroll=8)
        def _p1(i):
            f = buf[pl.ds(i * L, L)]
            k = _to_key(f)
            keys_v[pl.ds(i * L, L)] = k
            d12 = (k >> 20).astype(jnp.int32)
            plsc.addupdate_scatter(hist12, [d12], ones_i)

        b1, cum_gt1, cnt_eq1 = scan_hist(hist12, jnp.int32(K))

        binv = hist12[pl.ds(b1 * L, L)]
        rb = lax.rev(binv, (0,))
        rcum = plsc.cumsum(rb) + cum_gt1
        lane = _popcount(jnp.logical_not(rcum >= jnp.int32(K)))
        b12 = b1 * 16 + 15 - lane
        cnt_eq12 = jnp.sum(jnp.where(iota == lane, rb, 0))
        cum_gt12 = jnp.sum(jnp.where(iota == lane, rcum, 0)) - cnt_eq12

        @plsc.parallel_loop(0, N // L, unroll=8,
                            carry=(jnp.int32(0), jnp.int32(0)))
        def _c1(i, c):
            oA, oD = c
            k = keys_v[pl.ds(i * L, L)]
            d12 = (k >> 20).astype(jnp.int32)
            m_hi = d12 > b12
            m_eq = d12 == b12
            plsc.store_compressed(bufA.at[pl.ds(oA, L)], k, mask=m_hi)
            plsc.store_compressed(bufB.at[pl.ds(oD, L)], k, mask=m_eq)
            plsc.store_scatter(hist12, [d12], zeros_i)
            return oA + _popcount(m_hi), oD + _popcount(m_eq)

        offA, _ = _c1
        state = (offA, jnp.int32(K) - cum_gt12, cnt_eq12)

        state = select_level(bufB, bufC, 12, state)
        state = select_level(bufC, bufB, 4, state)
        state = select_level(bufB, bufC, 0, state)
        offA, need, cnt = state

        @pl.when(cnt > 0)
        def _():
            tvec = bufC[pl.ds(0, L)]

            def ap(jj, o):
                plsc.store_compressed(bufA.at[pl.ds(o, L)], tvec,
                                      mask=iota < need - jj * L)
                return o + jnp.minimum(need - jj * L, L)

            lax.fori_loop(0, (need + L - 1) // L, ap, offA)

        v = []
        for i in range(16):
            k = bufA[pl.ds(i * L, L)]
            ks, _ = plsc.sort_key_val(k, k, descending=True)
            v.append(ks)
        r = 1
        while r < 16:
            for base in range(0, 16, 2 * r):
                blk = v[base:base + r] + [
                    lax.rev(y, (0,))
                    for y in v[base + r:base + 2 * r][::-1]
                ]
                d = r
                while d >= 1:
                    nb = list(blk)
                    for b0 in range(0, 2 * r, 2 * d):
                        for i2 in range(d):
                            pa, pb = blk[b0 + i2], blk[b0 + i2 + d]
                            nb[b0 + i2] = jnp.maximum(pa, pb)
                            nb[b0 + i2 + d] = jnp.minimum(pa, pb)
                    blk = nb
                    d //= 2
                for i2 in range(2 * r):
                    ks, _ = plsc.sort_key_val(blk[i2], blk[i2],
                                              descending=True)
                    blk[i2] = ks
                v[base:base + 2 * r] = blk
            r *= 2
        for i in range(16):
            out_stage[pl.ds(j * K + i * L, L)] = _from_key(v[i])

    base_row = wid * ROWS_PER_W

    def in_copy(j, buf):
        return pltpu.make_async_copy(x_hbm.at[base_row + j], buf, sem)

    in_copy(0, row_v0).start()

    def pair(jj, _):
        j0 = jj * 2
        in_copy(j0, row_v0).wait()
        in_copy(j0 + 1, row_v1).start()
        process_row(row_v0, j0)
        in_copy(j0, row_v1).wait()
        in_copy(jnp.minimum(j0 + 2, ROWS_PER_W - 1), row_v0).start()
        process_row(row_v1, j0 + 1)
        return 0

    lax.fori_loop(0, ROWS_PER_W // 2, pair, 0)
    in_copy(0, row_v0).wait()
    pltpu.sync_copy(out_stage,
                    out_hbm.at[pl.ds(base_row * K, ROWS_PER_W * K)])


@jax.jit
def kernel(x):
    x2 = x.reshape(R, N)
    mesh = plsc.VectorSubcoreMesh(core_axis_name="c", subcore_axis_name="s",
                                  num_cores=NC, num_subcores=NS)
    out = pl.kernel(
        _body,
        out_type=jax.ShapeDtypeStruct((R * K,), jnp.float32),
        mesh=mesh,
        compiler_params=pltpu.CompilerParams(needs_layout_passes=False),
        scratch_types=[
            pltpu.VMEM((N,), jnp.float32),
            pltpu.VMEM((N,), jnp.float32),
            pltpu.VMEM((N,), jnp.uint32),
            pltpu.VMEM((K + 4 * L,), jnp.uint32),
            pltpu.VMEM((N + L,), jnp.uint32),
            pltpu.VMEM((N + L,), jnp.uint32),
            pltpu.VMEM((ROWS_PER_W * K,), jnp.float32),
            pltpu.VMEM((256 * L,), jnp.int32),
            pltpu.VMEM((4096,), jnp.int32),
            pltpu.SemaphoreType.DMA,
        ],
    )(x2)
    return out.reshape(64, 16, K)


# speedup vs baseline: 1.1379x; 1.1379x over previous
import functools

import numpy as np
import jax
import jax.numpy as jnp
from jax import lax
from jax.experimental import pallas as pl
from jax.experimental.pallas import tpu as pltpu
from jax.experimental.pallas import tpu_sc as plsc

L = 16
K = 256
N = 4096
R = 1024
NC = 2
NS = 16
NW = NC * NS
ROWS_PER_W = R // NW

_TOP = np.uint32(0x80000000)


def _to_key(f):
    b = plsc.bitcast(f, jnp.uint32)
    return jnp.where((b >> 31) == 1, ~b, b | _TOP)


def _from_key(k):
    b = jnp.where((k >> 31) == 1, k ^ _TOP, ~k)
    return plsc.bitcast(b, jnp.float32)


def _merge2_desc(v, w):
    rw = lax.rev(w, (0,))
    hi = jnp.maximum(v, rw)
    lo = jnp.minimum(v, rw)
    hi, _ = plsc.sort_key_val(hi, hi, descending=True)
    lo, _ = plsc.sort_key_val(lo, lo, descending=True)
    return hi, lo


def _popcount(m):
    return plsc.all_reduce_population_count(m)[0]


def _body(x_hbm, out_hbm, row_v0, row_v1, keys_v, bufA, bufB, bufC,
          out_stage, hist, hist12, sem):
    wid = lax.axis_index("c") * NS + lax.axis_index("s")
    iota = lax.iota(jnp.int32, L)
    zeros_i = jnp.zeros((L,), jnp.int32)
    ones_i = jnp.ones((L,), jnp.int32)

    def zero_hist(i, _):
        hist[pl.ds(i * L, L)] = zeros_i
        hist12[pl.ds((i % 256) * L, L)] = zeros_i
        return 0

    lax.fori_loop(0, 256, zero_hist, 0)

    def hist_pass(src, cnt, shift):
        @plsc.parallel_loop(0, (cnt + L - 1) // L, unroll=2)
        def _hp(i):
            k = src[pl.ds(i * L, L)]
            valid = iota < cnt - i * L
            d = ((k >> shift) & 0xFF).astype(jnp.int32)
            plsc.addupdate_scatter(hist, [d * L + iota], ones_i, mask=valid)

    def scan_hist(h, need):
        def gbody(jj, c):
            cum, found, gstar, cumg = c
            g = 15 - jj
            acc = h[pl.ds(g * 256, L)]
            for l in range(1, L):
                acc = acc + h[pl.ds(g * 256 + l * L, L)]
            gc = jnp.sum(acc)
            hit = jnp.logical_and(jnp.logical_not(found), cum + gc >= need)
            gstar = jnp.where(hit, g, gstar)
            cumg = jnp.where(hit, cum, cumg)
            found = jnp.logical_or(found, hit)
            cum = jnp.where(found, cum, cum + gc)
            return cum, found, gstar, cumg

        _, _, gstar, cumg = plsc.parallel_loop(
            0, 16, carry=(jnp.int32(0), jnp.bool_(False), jnp.int32(0),
                          jnp.int32(0)))(gbody)

        def ibody(l, c):
            cum, found, b, cnt = c
            d = gstar * 16 + 15 - l
            dc = jnp.sum(h[pl.ds(d * L, L)])
            hit = jnp.logical_and(jnp.logical_not(found), cum + dc >= need)
            b = jnp.where(hit, d, b)
            cnt = jnp.where(hit, dc, cnt)
            found = jnp.logical_or(found, hit)
            cum = jnp.where(found, cum, cum + dc)
            return cum, found, b, cnt

        cum_gt, _, b, cnt_eq = plsc.parallel_loop(
            0, 16, carry=(cumg, jnp.bool_(False), jnp.int32(0),
                          jnp.int32(0)))(ibody)
        return b, cum_gt, cnt_eq

    def compact_pass(src, dst, cnt, shift, bstar, offA):
        @plsc.parallel_loop(0, (cnt + L - 1) // L, unroll=2,
                            carry=(offA, jnp.int32(0)))
        def _cp(i, c):
            oA, oD = c
            k = src[pl.ds(i * L, L)]
            valid = iota < cnt - i * L
            d = ((k >> shift) & 0xFF).astype(jnp.int32)
            m_hi = jnp.logical_and(d > bstar, valid)
            m_eq = jnp.logical_and(d == bstar, valid)
            plsc.store_compressed(bufA.at[pl.ds(oA, L)], k, mask=m_hi)
            plsc.store_compressed(dst.at[pl.ds(oD, L)], k, mask=m_eq)
            plsc.store_scatter(hist, [d * L + iota], zeros_i, mask=valid)
            return oA + _popcount(m_hi), oD + _popcount(m_eq)

        return _cp

    def select_level(src, dst, shift, state):
        def skip(args):
            return args

        def run(args):
            def small(args2):
                offA, need, cnt = args2
                vs = []
                for j in range(3):
                    k = src[pl.ds(j * L, L)]
                    kv = jnp.where(iota + j * L < cnt, k, jnp.uint32(0))
                    ks, _ = plsc.sort_key_val(kv, kv, descending=True)
                    vs.append(ks)
                a0, a1 = _merge2_desc(vs[0], vs[1])
                zpad = jnp.zeros((L,), jnp.uint32)
                b0, b1r = lax.rev(zpad, (0,)), lax.rev(vs[2], (0,))
                t0 = jnp.maximum(a0, b0)
                t2 = jnp.minimum(a0, b0)
                t1 = jnp.maximum(a1, b1r)
                t3 = jnp.minimum(a1, b1r)
                outs = []
                for w in (jnp.maximum(t0, t1), jnp.minimum(t0, t1),
                          jnp.maximum(t2, t3)):
                    ws, _ = plsc.sort_key_val(w, w, descending=True)
                    outs.append(ws)
                for j in range(3):
                    plsc.store_compressed(bufA.at[pl.ds(offA + j * L, L)],
                                          outs[j],
                                          mask=iota < need - j * L)
                return offA + need, jnp.int32(0), jnp.int32(0)

            def big(args2):
                offA, need, cnt = args2
                hist_pass(src, cnt, shift)
                b, cum_gt, cnt_eq = scan_hist(hist, need)
                offA, _ = compact_pass(src, dst, cnt, shift, b, offA)
                return offA, need - cum_gt, cnt_eq

            return lax.cond(args[2] <= 3 * L, small, big, args)

        return lax.cond(state[2] > 0, run, skip, state)

    def process_row(buf, j):
        @plsc.parallel_loop(0, N // L, unroll=4)
        def _p1(i):
            f = buf[pl.ds(i * L, L)]
            k = _to_key(f)
            keys_v[pl.ds(i * L, L)] = k
            d12 = (k >> 20).astype(jnp.int32)
            plsc.addupdate_scatter(hist, [(d12 >> 4) * L + iota], ones_i)
            plsc.addupdate_scatter(hist12, [d12], ones_i)

        b1, cum_gt1, cnt_eq1 = scan_hist(hist, jnp.int32(K))

        binv = hist12[pl.ds(b1 * L, L)]
        rb = lax.rev(binv, (0,))
        rcum = plsc.cumsum(rb) + cum_gt1
        lane = _popcount(jnp.logical_not(rcum >= jnp.int32(K)))
        b12 = b1 * 16 + 15 - lane
        cnt_eq12 = jnp.sum(jnp.where(iota == lane, rb, 0))
        cum_gt12 = jnp.sum(jnp.where(iota == lane, rcum, 0)) - cnt_eq12

        @plsc.parallel_loop(0, N // L, unroll=4,
                            carry=(jnp.int32(0), jnp.int32(0)))
        def _c1(i, c):
            oA, oD = c
            k = keys_v[pl.ds(i * L, L)]
            d12 = (k >> 20).astype(jnp.int32)
            m_hi = d12 > b12
            m_eq = d12 == b12
            plsc.store_compressed(bufA.at[pl.ds(oA, L)], k, mask=m_hi)
            plsc.store_compressed(bufB.at[pl.ds(oD, L)], k, mask=m_eq)
            plsc.store_scatter(hist, [(d12 >> 4) * L + iota], zeros_i)
            plsc.store_scatter(hist12, [d12], zeros_i)
            return oA + _popcount(m_hi), oD + _popcount(m_eq)

        offA, _ = _c1
        state = (offA, jnp.int32(K) - cum_gt12, cnt_eq12)

        state = select_level(bufB, bufC, 12, state)
        state = select_level(bufC, bufB, 4, state)
        state = select_level(bufB, bufC, 0, state)
        offA, need, cnt = state

        @pl.when(cnt > 0)
        def _():
            tvec = bufC[pl.ds(0, L)]

            def ap(jj, o):
                plsc.store_compressed(bufA.at[pl.ds(o, L)], tvec,
                                      mask=iota < need - jj * L)
                return o + jnp.minimum(need - jj * L, L)

            lax.fori_loop(0, (need + L - 1) // L, ap, offA)

        v = []
        for i in range(16):
            k = bufA[pl.ds(i * L, L)]
            ks, _ = plsc.sort_key_val(k, k, descending=True)
            v.append(ks)
        r = 1
        while r < 16:
            for base in range(0, 16, 2 * r):
                blk = v[base:base + r] + [
                    lax.rev(y, (0,))
                    for y in v[base + r:base + 2 * r][::-1]
                ]
                d = r
                while d >= 1:
                    nb = list(blk)
                    for b0 in range(0, 2 * r, 2 * d):
                        for i2 in range(d):
                            pa, pb = blk[b0 + i2], blk[b0 + i2 + d]
                            nb[b0 + i2] = jnp.maximum(pa, pb)
                            nb[b0 + i2 + d] = jnp.minimum(pa, pb)
                    blk = nb
                    d //= 2
                for i2 in range(2 * r):
                    ks, _ = plsc.sort_key_val(blk[i2], blk[i2],
                                              descending=True)
                    blk[i2] = ks
                v[base:base + 2 * r] = blk
            r *= 2
        for i in range(16):
            out_stage[pl.ds(j * K + i * L, L)] = _from_key(v[i])

    base_row = wid * ROWS_PER_W

    def in_copy(j, buf):
        return pltpu.make_async_copy(x_hbm.at[base_row + j], buf, sem)

    in_copy(0, row_v0).start()

    def pair(jj, _):
        j0 = jj * 2
        in_copy(j0, row_v0).wait()
        in_copy(j0 + 1, row_v1).start()
        process_row(row_v0, j0)
        in_copy(j0, row_v1).wait()
        in_copy(jnp.minimum(j0 + 2, ROWS_PER_W - 1), row_v0).start()
        process_row(row_v1, j0 + 1)
        return 0

    lax.fori_loop(0, ROWS_PER_W // 2, pair, 0)
    in_copy(0, row_v0).wait()
    pltpu.sync_copy(out_stage,
                    out_hbm.at[pl.ds(base_row * K, ROWS_PER_W * K)])


@jax.jit
def kernel(x):
    x2 = x.reshape(R, N)
    mesh = plsc.VectorSubcoreMesh(core_axis_name="c", subcore_axis_name="s",
                                  num_cores=NC, num_subcores=NS)
    out = pl.kernel(
        _body,
        out_type=jax.ShapeDtypeStruct((R * K,), jnp.float32),
        mesh=mesh,
        compiler_params=pltpu.CompilerParams(needs_layout_passes=False),
        scratch_types=[
            pltpu.VMEM((N,), jnp.float32),
            pltpu.VMEM((N,), jnp.float32),
            pltpu.VMEM((N,), jnp.uint32),
            pltpu.VMEM((K + 4 * L,), jnp.uint32),
            pltpu.VMEM((N + L,), jnp.uint32),
            pltpu.VMEM((N + L,), jnp.uint32),
            pltpu.VMEM((ROWS_PER_W * K,), jnp.float32),
            pltpu.VMEM((256 * L,), jnp.int32),
            pltpu.VMEM((4096,), jnp.int32),
            pltpu.SemaphoreType.DMA,
        ],
    )(x2)
    return out.reshape(64, 16, K)
